# baseline (speedup 1.0000x reference)
import functools

import jax
import jax.numpy as jnp
from jax.experimental import pallas as pl
from jax.experimental.pallas import tpu as pltpu


def _ceil_to(x, m):
    return (x + m - 1) // m * m


def _edgeconv_layer1(xb_ref, xd_ref, adj_ref, w_ref, b_ref,
                     h1f_ref, h1b_ref, inv_ref, msk_ref):
    deg = jnp.sum(adj_ref[...].astype(jnp.float32), axis=1, keepdims=True)
    inv = 1.0 / jnp.maximum(deg, 1.0)
    msk = (deg > 0).astype(jnp.float32)
    inv_ref[...] = inv
    msk_ref[...] = msk
    agg = jnp.dot(adj_ref[...], xb_ref[...],
                  preferred_element_type=jnp.float32) * inv
    pre = jnp.concatenate([xd_ref[...], agg], axis=1)
    h = jnp.dot(pre, w_ref[...], preferred_element_type=jnp.float32) + b_ref[...]
    h = jnp.maximum(h, 0.0) * msk
    h1f_ref[...] = h
    h1b_ref[...] = h.astype(jnp.bfloat16)


def _edgeconv_layer2_final(h1b_ref, h1d_ref, xd_ref, adj_ref, w_ref, b_ref,
                           wf_ref, bf_ref, inv_ref, msk_ref, o_ref):
    agg = jnp.dot(adj_ref[...], h1b_ref[...],
                  preferred_element_type=jnp.float32) * inv_ref[...]
    pre = jnp.concatenate([h1d_ref[...], agg], axis=1)
    h2 = jnp.dot(pre, w_ref[...], preferred_element_type=jnp.float32) + b_ref[...]
    h2 = jnp.maximum(h2, 0.0) * msk_ref[...]
    fin = jnp.concatenate([xd_ref[...], h1d_ref[...], h2], axis=1)
    o_ref[...] = (jnp.dot(fin, wf_ref[...], preferred_element_type=jnp.float32)
                  + bf_ref[...])


def _pack_conv_weights(W, c_prev):
    W1, W2 = W[:, :c_prev], W[:, c_prev:]
    return jnp.concatenate([(W1 - W2).T, W2.T], axis=0)


@jax.jit
def kernel(x, edge_index, W0, b0, W1, b1, Wf, bf):
    n, c_in = x.shape
    c_mid = W0.shape[0]
    out_dim = Wf.shape[0]

    TM = 256
    n_pad = _ceil_to(n, TM)
    grid = (n_pad // TM,)

    src, dst = edge_index[0], edge_index[1]
    adj = jnp.zeros((n_pad, n_pad), jnp.int8).at[dst, src].add(jnp.int8(1))
    adj = adj.astype(jnp.bfloat16)

    xf = jnp.pad(x, ((0, n_pad - n), (0, 0))) if n_pad != n else x
    xb = xf.astype(jnp.bfloat16)

    w1 = _pack_conv_weights(W0, c_in)
    w2 = _pack_conv_weights(W1, c_mid)
    wf_t = Wf.T

    compiler_params = pltpu.CompilerParams(
        dimension_semantics=("parallel",),
        vmem_limit_bytes=64 * 1024 * 1024,
    )

    def full(a):
        return pl.BlockSpec(a.shape, lambda i: (0, 0))

    def row_tile(c, dt=None):
        return pl.BlockSpec((TM, c), lambda i: (i, 0))

    adj_spec = pl.BlockSpec((TM, n_pad), lambda i: (i, 0))
    vec_spec = pl.BlockSpec((TM, 1), lambda i: (i, 0))

    h1f, h1b, inv, msk = pl.pallas_call(
        _edgeconv_layer1,
        out_shape=(jax.ShapeDtypeStruct((n_pad, c_mid), jnp.float32),
                   jax.ShapeDtypeStruct((n_pad, c_mid), jnp.bfloat16),
                   jax.ShapeDtypeStruct((n_pad, 1), jnp.float32),
                   jax.ShapeDtypeStruct((n_pad, 1), jnp.float32)),
        grid=grid,
        in_specs=[full(xb), row_tile(c_in), adj_spec,
                  full(w1), pl.BlockSpec((1, c_mid), lambda i: (0, 0))],
        out_specs=(row_tile(c_mid), row_tile(c_mid), vec_spec, vec_spec),
        compiler_params=compiler_params,
    )(xb, xf, adj, w1, b0.reshape(1, -1))

    out = pl.pallas_call(
        _edgeconv_layer2_final,
        out_shape=jax.ShapeDtypeStruct((n_pad, out_dim), jnp.float32),
        grid=grid,
        in_specs=[full(h1b), row_tile(c_mid), row_tile(c_in), adj_spec,
                  full(w2), pl.BlockSpec((1, c_mid), lambda i: (0, 0)),
                  full(wf_t), pl.BlockSpec((1, out_dim), lambda i: (0, 0)),
                  vec_spec, vec_spec],
        out_specs=row_tile(out_dim),
        compiler_params=compiler_params,
    )(h1b, h1f, xf, adj, w2, b1.reshape(1, -1), wf_t, bf.reshape(1, -1),
      inv, msk)

    return out[:n]


# speedup vs baseline: 1.6222x; 1.6222x over previous
import functools

import jax
import jax.numpy as jnp
from jax.experimental import pallas as pl
from jax.experimental.pallas import tpu as pltpu


def _ceil_to(x, m):
    return (x + m - 1) // m * m


def _edgeconv_layer1(x_ref, adj_ref, w_ref, b_ref,
                     h1_ref, inv_ref, msk_ref):
    tm = h1_ref.shape[0]
    row0 = pl.multiple_of(pl.program_id(0) * tm, tm)
    adj = adj_ref[...]
    deg = jnp.sum(adj, axis=1, keepdims=True)
    inv = 1.0 / jnp.maximum(deg, 1.0)
    msk = (deg > 0).astype(jnp.float32)
    inv_ref[...] = inv
    msk_ref[...] = msk
    agg = jnp.dot(adj, x_ref[...], preferred_element_type=jnp.float32) * inv
    pre = jnp.concatenate([x_ref[pl.ds(row0, tm), :], agg], axis=1)
    h = jnp.dot(pre, w_ref[...], preferred_element_type=jnp.float32) + b_ref[...]
    h1_ref[...] = jnp.maximum(h, 0.0) * msk


def _edgeconv_layer2_final(x_ref, h1_ref, adj_ref, w_ref, b_ref,
                           wf_ref, bf_ref, inv_ref, msk_ref, o_ref):
    tm = o_ref.shape[0]
    row0 = pl.multiple_of(pl.program_id(0) * tm, tm)
    agg = (jnp.dot(adj_ref[...], h1_ref[...],
                   preferred_element_type=jnp.float32) * inv_ref[...])
    h1d = h1_ref[pl.ds(row0, tm), :]
    pre = jnp.concatenate([h1d, agg], axis=1)
    h2 = jnp.dot(pre, w_ref[...], preferred_element_type=jnp.float32) + b_ref[...]
    h2 = jnp.maximum(h2, 0.0) * msk_ref[...]
    fin = jnp.concatenate([x_ref[pl.ds(row0, tm), :], h1d, h2], axis=1)
    o_ref[...] = (jnp.dot(fin, wf_ref[...], preferred_element_type=jnp.float32)
                  + bf_ref[...])


def _pack_conv_weights(W, c_prev):
    W1, W2 = W[:, :c_prev], W[:, c_prev:]
    return jnp.concatenate([(W1 - W2).T, W2.T], axis=0)


@jax.jit
def kernel(x, edge_index, W0, b0, W1, b1, Wf, bf):
    n, c_in = x.shape
    c_mid = W0.shape[0]
    out_dim = Wf.shape[0]

    TM = 256
    n_pad = _ceil_to(n, TM)
    grid = (n_pad // TM,)

    src, dst = edge_index[0], edge_index[1]
    lin = dst * n_pad + src
    adj = (jnp.zeros((n_pad * n_pad,), jnp.float32)
           .at[lin].add(1.0)
           .reshape(n_pad, n_pad))

    xf = jnp.pad(x, ((0, n_pad - n), (0, 0))) if n_pad != n else x

    w1 = _pack_conv_weights(W0, c_in)
    w2 = _pack_conv_weights(W1, c_mid)
    wf_t = Wf.T

    compiler_params = pltpu.CompilerParams(
        dimension_semantics=("parallel",),
        vmem_limit_bytes=60 * 1024 * 1024,
    )

    def full(a):
        return pl.BlockSpec(a.shape, lambda i: (0, 0))

    def row_tile(c):
        return pl.BlockSpec((TM, c), lambda i: (i, 0))

    adj_spec = pl.BlockSpec((TM, n_pad), lambda i: (i, 0))
    vec_spec = pl.BlockSpec((TM, 1), lambda i: (i, 0))

    h1, inv, msk = pl.pallas_call(
        _edgeconv_layer1,
        out_shape=(jax.ShapeDtypeStruct((n_pad, c_mid), jnp.float32),
                   jax.ShapeDtypeStruct((n_pad, 1), jnp.float32),
                   jax.ShapeDtypeStruct((n_pad, 1), jnp.float32)),
        grid=grid,
        in_specs=[full(xf), adj_spec,
                  full(w1), pl.BlockSpec((1, c_mid), lambda i: (0, 0))],
        out_specs=(row_tile(c_mid), vec_spec, vec_spec),
        compiler_params=compiler_params,
    )(xf, adj, w1, b0.reshape(1, -1))

    out = pl.pallas_call(
        _edgeconv_layer2_final,
        out_shape=jax.ShapeDtypeStruct((n_pad, out_dim), jnp.float32),
        grid=grid,
        in_specs=[full(xf), full(h1), adj_spec,
                  full(w2), pl.BlockSpec((1, c_mid), lambda i: (0, 0)),
                  full(wf_t), pl.BlockSpec((1, out_dim), lambda i: (0, 0)),
                  vec_spec, vec_spec],
        out_specs=row_tile(out_dim),
        compiler_params=compiler_params,
    )(xf, h1, adj, w2, b1.reshape(1, -1), wf_t, bf.reshape(1, -1), inv, msk)

    return out[:n]
